# baseline (speedup 1.0000x reference)
import jax
from jax.experimental import pallas as pl
from jax.experimental.pallas import tpu as pltpu

_ROWS = 256


def _copy_body(in_ref, out_ref):
    out_ref[...] = in_ref[...]


def kernel(inputs):
    shape = inputs.shape
    flat = inputs.reshape(-1, shape[-1])
    n_rows, n_cols = flat.shape
    grid = (n_rows // _ROWS,)
    out = pl.pallas_call(
        _copy_body,
        out_shape=jax.ShapeDtypeStruct(flat.shape, flat.dtype),
        grid=grid,
        in_specs=[pl.BlockSpec((_ROWS, n_cols), lambda i: (i, 0))],
        out_specs=pl.BlockSpec((_ROWS, n_cols), lambda i: (i, 0)),
        compiler_params=pltpu.CompilerParams(
            dimension_semantics=("parallel",),
        ),
    )(flat)
    return out.reshape(shape)
